# Initial kernel scaffold; baseline (speedup 1.0000x reference)
import jax
import jax.numpy as jnp
from jax.experimental import pallas as pl


def kernel(x, edge_index, W1, b1, W2, b2):
    raise NotImplementedError("write your pallas kernel here")


# speedup vs baseline: 28.0722x; 28.0722x over previous
import functools

import jax
import jax.numpy as jnp
from jax import lax
from jax.experimental import pallas as pl
from jax.experimental.pallas import tpu as pltpu
from jax.experimental.pallas import tpu_sc as plsc

N = 10000
DI = 128
DH = 128
DO = 40
E = 320000

NC = 2
NS = 16
NW = NC * NS
CH = 128
NCHUNK = 80
EP = NW * NCHUNK * CH
NP = 10240
PAD_ROWS = NP - N
RPW = NP // NS

_mesh = plsc.VectorSubcoreMesh(core_axis_name="c", subcore_axis_name="s")


def _make_deg():
    @functools.partial(
        pl.kernel,
        out_type=jax.ShapeDtypeStruct((NC * NP, 16), jnp.float32),
        mesh=_mesh,
        scratch_types=[
            pltpu.VMEM((NCHUNK, CH), jnp.int32),
            pltpu.VMEM((CH, 16), jnp.float32),
            pltpu.VMEM_SHARED((NP, 16), jnp.float32),
        ],
        compiler_params=pltpu.CompilerParams(use_tc_tiling_on_sc=False),
    )
    def deg_kernel(dst_hbm, ones_hbm, zeros_hbm, out_hbm, dst_v, ones_v, deg_sp):
        cid = lax.axis_index("c")
        sid = lax.axis_index("s")
        wid = cid * NS + sid
        pltpu.sync_copy(dst_hbm.at[wid], dst_v)
        pltpu.sync_copy(ones_hbm, ones_v)
        pltpu.sync_copy(zeros_hbm.at[pl.ds(sid * RPW, RPW)],
                        deg_sp.at[pl.ds(sid * RPW, RPW)])
        plsc.subcore_barrier()

        def body(j, _):
            pltpu.sync_copy(ones_v, deg_sp.at[dst_v.at[j]], add=True)
            return ()

        lax.fori_loop(0, NCHUNK, body, (), unroll=False)
        plsc.subcore_barrier()
        pltpu.sync_copy(deg_sp.at[pl.ds(sid * RPW, RPW)],
                        out_hbm.at[pl.ds(cid * NP + sid * RPW, RPW)])

    return deg_kernel


def _make_agg(d):
    @functools.partial(
        pl.kernel,
        out_type=jax.ShapeDtypeStruct((NC * NP, d), jnp.float32),
        mesh=_mesh,
        scratch_types=[
            pltpu.VMEM((NCHUNK // 2, CH), jnp.int32),
            pltpu.VMEM((NCHUNK // 2, CH), jnp.int32),
            pltpu.VMEM((CH, d), jnp.float32),
            pltpu.VMEM((CH, d), jnp.float32),
            pltpu.VMEM_SHARED((NP, d), jnp.float32),
            pltpu.SemaphoreType.DMA,
            pltpu.SemaphoreType.DMA,
        ],
        compiler_params=pltpu.CompilerParams(use_tc_tiling_on_sc=False),
    )
    def agg_kernel(table_hbm, src_hbm, dst_hbm, zeros_hbm, out_hbm,
                   src_v, dst_v, buf0, buf1, acc_sp, sem0, sem1):
        cid = lax.axis_index("c")
        sid = lax.axis_index("s")
        wid = cid * NS + sid
        hc = NCHUNK // 2
        pltpu.sync_copy(zeros_hbm.at[pl.ds(sid * RPW, RPW)],
                        acc_sp.at[pl.ds(sid * RPW, RPW)])
        plsc.subcore_barrier()

        def wait(sem, buf):
            pltpu.make_async_copy(table_hbm.at[pl.ds(0, CH)], buf, sem).wait()

        for h in range(2):
            pltpu.sync_copy(src_hbm.at[wid, pl.ds(h * hc, hc)], src_v)
            pltpu.sync_copy(dst_hbm.at[wid, pl.ds(h * hc, hc)], dst_v)
            pltpu.async_copy(table_hbm.at[src_v.at[0]], buf0, sem0)

            def body(jj, _):
                j = jj * 2
                pltpu.async_copy(table_hbm.at[src_v.at[j + 1]], buf1, sem1)
                wait(sem0, buf0)
                pltpu.sync_copy(buf0, acc_sp.at[dst_v.at[j]], add=True)

                @pl.when(j + 2 < hc)
                def _():
                    pltpu.async_copy(table_hbm.at[src_v.at[j + 2]], buf0, sem0)

                wait(sem1, buf1)
                pltpu.sync_copy(buf1, acc_sp.at[dst_v.at[j + 1]], add=True)
                return ()

            lax.fori_loop(0, hc // 2, body, (), unroll=False)
        plsc.subcore_barrier()
        pltpu.sync_copy(acc_sp.at[pl.ds(sid * RPW, RPW)],
                        out_hbm.at[pl.ds(cid * NP + sid * RPW, RPW)])

    return agg_kernel


_deg_call = _make_deg()
_agg128 = _make_agg(DH)
_agg40 = _make_agg(DO)

BR = 256


def _tc1_body(deg0_ref, deg1_ref, x_ref, w1_ref, hs_ref, dis_ref):
    deg = deg0_ref[:, 0:1] + deg1_ref[:, 0:1] + 1.0
    disb = jnp.broadcast_to(lax.rsqrt(deg), (BR, DH))
    h = jnp.dot(x_ref[...], w1_ref[...], preferred_element_type=jnp.float32)
    hs_ref[...] = disb * h
    dis_ref[...] = disb


def _tc2_body(acc0_ref, acc1_ref, hs1_ref, dis_ref, b1_ref, w2_ref, hs2_ref):
    agg = (acc0_ref[...] + acc1_ref[...] + hs1_ref[...]) * dis_ref[...]
    m1 = jnp.maximum(agg + b1_ref[...], 0.0)
    h2 = jnp.dot(m1, w2_ref[...], preferred_element_type=jnp.float32)
    hs2_ref[...] = dis_ref[:, 0:DO] * h2


def _tc3_body(acc0_ref, acc1_ref, hs2_ref, dis_ref, b2_ref, out_ref):
    agg = (acc0_ref[...] + acc1_ref[...] + hs2_ref[...]) * dis_ref[:, 0:DO]
    out_ref[...] = agg + b2_ref[...]


def kernel(x, edge_index, W1, b1, W2, b2):
    ei = edge_index.astype(jnp.int32)
    pad = N + (jnp.arange(EP - E, dtype=jnp.int32) % PAD_ROWS)
    src3 = jnp.concatenate([ei[0], pad]).reshape(NW, NCHUNK, CH)
    dst3 = jnp.concatenate([ei[1], pad]).reshape(NW, NCHUNK, CH)
    xp = jnp.zeros((NP, DI), jnp.float32).at[:N].set(x)
    ones16 = jnp.ones((CH, 16), jnp.float32)
    z16 = jnp.zeros((NP, 16), jnp.float32)
    zH = jnp.zeros((NP, DH), jnp.float32)
    zO = jnp.zeros((NP, DO), jnp.float32)

    degs = _deg_call(dst3, ones16, z16)

    grid = (NP // BR,)
    row = lambda w: pl.BlockSpec((BR, w), lambda i: (i, 0))
    full = lambda a, b: pl.BlockSpec((a, b), lambda i: (0, 0))

    hs1, dis = pl.pallas_call(
        _tc1_body,
        grid=grid,
        in_specs=[row(16), row(16), row(DI), full(DI, DH)],
        out_specs=[row(DH), row(DH)],
        out_shape=[jax.ShapeDtypeStruct((NP, DH), jnp.float32),
                   jax.ShapeDtypeStruct((NP, DH), jnp.float32)],
    )(degs[:NP], degs[NP:], xp, W1)

    acc1 = _agg128(hs1, src3, dst3, zH)

    hs2 = pl.pallas_call(
        _tc2_body,
        grid=grid,
        in_specs=[row(DH), row(DH), row(DH), row(DH), full(1, DH), full(DH, DO)],
        out_specs=row(DO),
        out_shape=jax.ShapeDtypeStruct((NP, DO), jnp.float32),
    )(acc1[:NP], acc1[NP:], hs1, dis, b1.reshape(1, DH), W2)

    acc2 = _agg40(hs2, src3, dst3, zO)

    outp = pl.pallas_call(
        _tc3_body,
        grid=grid,
        in_specs=[row(DO), row(DO), row(DO), row(DH), full(1, DO)],
        out_specs=row(DO),
        out_shape=jax.ShapeDtypeStruct((NP, DO), jnp.float32),
    )(acc2[:NP], acc2[NP:], hs2, dis, b2.reshape(1, DO))

    return outp[:N]


# speedup vs baseline: 29.8827x; 1.0645x over previous
import functools

import jax
import jax.numpy as jnp
from jax import lax
from jax.experimental import pallas as pl
from jax.experimental.pallas import tpu as pltpu
from jax.experimental.pallas import tpu_sc as plsc

N = 10000
DI = 128
DH = 128
DO = 40
E = 320000

NC = 2
NS = 16
NW = NC * NS
CH = 128
NCH = E // CH
CPW = NCH // NW
NTAIL = NCH - CPW * NW
HC = CPW // 2
RPW = N // NS

_mesh = plsc.VectorSubcoreMesh(core_axis_name="c", subcore_axis_name="s")
_sc_params = pltpu.CompilerParams(use_tc_tiling_on_sc=False)


def _zero_rows(buf, d):
    zero = jnp.zeros((16,), jnp.float32)

    def body(i, _):
        for t in range(-(-d // 16)):
            buf[i, pl.ds(min(16 * t, d - 16), 16)] = zero
        return ()

    lax.fori_loop(0, CH, body, (), unroll=False)


def _zero_spmem_slice(buf, table, sid, d):
    base = sid * RPW
    for k in range(4):
        pltpu.sync_copy(buf, table.at[pl.ds(base + k * CH, CH)])
    pltpu.sync_copy(buf.at[pl.ds(0, RPW - 4 * CH)],
                    table.at[pl.ds(base + 4 * CH, RPW - 4 * CH)])


def _make_deg():
    @functools.partial(
        pl.kernel,
        out_type=jax.ShapeDtypeStruct((NC, N, 16), jnp.float32),
        mesh=_mesh,
        scratch_types=[
            pltpu.VMEM((HC, CH), jnp.int32),
            pltpu.VMEM((1, CH), jnp.int32),
            pltpu.VMEM((CH, 16), jnp.float32),
            pltpu.VMEM((CH, 16), jnp.float32),
            pltpu.VMEM_SHARED((N, 16), jnp.float32),
        ],
        compiler_params=_sc_params,
    )
    def deg_kernel(e_hbm, out_hbm, dst_v, tdst_v, ones_v, zbuf, deg_sp):
        cid = lax.axis_index("c")
        sid = lax.axis_index("s")
        wid = cid * NS + sid
        one = jnp.ones((16,), jnp.float32)

        def fill_ones(i, _):
            ones_v[i] = one
            return ()

        lax.fori_loop(0, CH, fill_ones, (), unroll=False)
        _zero_rows(zbuf, 16)
        _zero_spmem_slice(zbuf, deg_sp, sid, 16)
        plsc.subcore_barrier()

        for h in range(2):
            pltpu.sync_copy(e_hbm.at[1, pl.ds(wid * CPW + h * HC, HC)], dst_v)

            def body(j, _):
                pltpu.sync_copy(ones_v, deg_sp.at[dst_v.at[j]], add=True)
                return ()

            lax.fori_loop(0, HC, body, (), unroll=False)

        @pl.when(wid < NTAIL)
        def _():
            pltpu.sync_copy(e_hbm.at[1, pl.ds(NW * CPW + wid, 1)], tdst_v)
            pltpu.sync_copy(ones_v, deg_sp.at[tdst_v.at[0]], add=True)

        plsc.subcore_barrier()
        pltpu.sync_copy(deg_sp.at[pl.ds(sid * RPW, RPW)],
                        out_hbm.at[cid, pl.ds(sid * RPW, RPW)])

    return deg_kernel


def _make_agg(d):
    @functools.partial(
        pl.kernel,
        out_type=jax.ShapeDtypeStruct((NC, N, d), jnp.float32),
        mesh=_mesh,
        scratch_types=[
            pltpu.VMEM((HC, CH), jnp.int32),
            pltpu.VMEM((HC, CH), jnp.int32),
            pltpu.VMEM((1, CH), jnp.int32),
            pltpu.VMEM((1, CH), jnp.int32),
            pltpu.VMEM((CH, d), jnp.float32),
            pltpu.VMEM((CH, d), jnp.float32),
            pltpu.VMEM_SHARED((N, d), jnp.float32),
            pltpu.SemaphoreType.DMA,
            pltpu.SemaphoreType.DMA,
            pltpu.SemaphoreType.DMA,
            pltpu.SemaphoreType.DMA,
        ],
        compiler_params=_sc_params,
    )
    def agg_kernel(table_hbm, e_hbm, out_hbm, src_v, dst_v, tsrc_v, tdst_v,
                   buf0, buf1, acc_sp, g0, g1, s0, s1):
        cid = lax.axis_index("c")
        sid = lax.axis_index("s")
        wid = cid * NS + sid

        def gwait(sem, buf):
            pltpu.make_async_copy(table_hbm.at[pl.ds(0, CH)], buf, sem).wait()

        def swait(sem, buf):
            pltpu.make_async_copy(table_hbm.at[pl.ds(0, CH)], buf, sem).wait()

        _zero_rows(buf0, d)
        _zero_spmem_slice(buf0, acc_sp, sid, d)
        plsc.subcore_barrier()

        for h in range(2):
            base = wid * CPW + h * HC
            pltpu.sync_copy(e_hbm.at[0, pl.ds(base, HC)], src_v)
            pltpu.sync_copy(e_hbm.at[1, pl.ds(base, HC)], dst_v)
            pltpu.async_copy(table_hbm.at[src_v.at[0]], buf0, g0)
            pltpu.async_copy(table_hbm.at[src_v.at[1]], buf1, g1)

            def body(jj, _):
                j = jj * 2
                gwait(g0, buf0)
                pltpu.async_copy(buf0, acc_sp.at[dst_v.at[j]], s0, add=True)
                gwait(g1, buf1)
                pltpu.async_copy(buf1, acc_sp.at[dst_v.at[j + 1]], s1, add=True)
                swait(s0, buf0)
                pltpu.async_copy(table_hbm.at[src_v.at[j + 2]], buf0, g0)

                @pl.when(j + 3 < HC)
                def _():
                    swait(s1, buf1)
                    pltpu.async_copy(table_hbm.at[src_v.at[j + 3]], buf1, g1)

                return ()

            lax.fori_loop(0, (HC - 1) // 2, body, (), unroll=False)
            gwait(g0, buf0)
            pltpu.sync_copy(buf0, acc_sp.at[dst_v.at[HC - 1]], add=True)
            swait(s1, buf1)

        @pl.when(wid < NTAIL)
        def _():
            pltpu.sync_copy(e_hbm.at[0, pl.ds(NW * CPW + wid, 1)], tsrc_v)
            pltpu.sync_copy(e_hbm.at[1, pl.ds(NW * CPW + wid, 1)], tdst_v)
            pltpu.async_copy(table_hbm.at[tsrc_v.at[0]], buf0, g0)
            gwait(g0, buf0)
            pltpu.sync_copy(buf0, acc_sp.at[tdst_v.at[0]], add=True)

        plsc.subcore_barrier()
        pltpu.sync_copy(acc_sp.at[pl.ds(sid * RPW, RPW)],
                        out_hbm.at[cid, pl.ds(sid * RPW, RPW)])

    return agg_kernel


_deg_call = _make_deg()
_agg128 = _make_agg(DH)
_agg40 = _make_agg(DO)

BR = 400


def _mm_body(x_ref, w1_ref, h_ref):
    h_ref[...] = jnp.dot(x_ref[...], w1_ref[...],
                         preferred_element_type=jnp.float32)


def _dis(deg_ref):
    deg = deg_ref[0, :, 0:1] + deg_ref[1, :, 0:1] + 1.0
    return lax.rsqrt(deg)


def _scale_body(deg_ref, h_ref, hs_ref):
    hs_ref[...] = _dis(deg_ref) * h_ref[...]


def _tc2_body(deg_ref, acc_ref, hs1_ref, b1_ref, w2_ref, hs2_ref):
    dis = _dis(deg_ref)
    agg = (acc_ref[0] + acc_ref[1] + hs1_ref[...]) * dis
    m1 = jnp.maximum(agg + b1_ref[...], 0.0)
    h2 = jnp.dot(m1, w2_ref[...], preferred_element_type=jnp.float32)
    hs2_ref[...] = dis * h2


def _tc3_body(deg_ref, acc_ref, hs2_ref, b2_ref, out_ref):
    agg = (acc_ref[0] + acc_ref[1] + hs2_ref[...]) * _dis(deg_ref)
    out_ref[...] = agg + b2_ref[...]


def kernel(x, edge_index, W1, b1, W2, b2):
    e3 = edge_index.astype(jnp.int32).reshape(2, NCH, CH)

    grid = (N // BR,)
    row = lambda w: pl.BlockSpec((BR, w), lambda i: (i, 0))
    deg_spec = pl.BlockSpec((2, BR, 16), lambda i: (0, i, 0))
    acc_spec = lambda w: pl.BlockSpec((2, BR, w), lambda i: (0, i, 0))
    full = lambda a, b: pl.BlockSpec((a, b), lambda i: (0, 0))

    degs = _deg_call(e3)

    h1 = pl.pallas_call(
        _mm_body,
        grid=grid,
        in_specs=[row(DI), full(DI, DH)],
        out_specs=row(DH),
        out_shape=jax.ShapeDtypeStruct((N, DH), jnp.float32),
    )(x, W1)

    hs1 = pl.pallas_call(
        _scale_body,
        grid=grid,
        in_specs=[deg_spec, row(DH)],
        out_specs=row(DH),
        out_shape=jax.ShapeDtypeStruct((N, DH), jnp.float32),
    )(degs, h1)

    acc1 = _agg128(hs1, e3)

    hs2 = pl.pallas_call(
        _tc2_body,
        grid=grid,
        in_specs=[deg_spec, acc_spec(DH), row(DH), full(1, DH), full(DH, DO)],
        out_specs=row(DO),
        out_shape=jax.ShapeDtypeStruct((N, DO), jnp.float32),
    )(degs, acc1, hs1, b1.reshape(1, DH), W2)

    acc2 = _agg40(hs2, e3)

    out = pl.pallas_call(
        _tc3_body,
        grid=grid,
        in_specs=[deg_spec, acc_spec(DO), row(DO), full(1, DO)],
        out_specs=row(DO),
        out_shape=jax.ShapeDtypeStruct((N, DO), jnp.float32),
    )(degs, acc2, hs2, b2.reshape(1, DO))

    return out


# speedup vs baseline: 37.8182x; 1.2656x over previous
import functools

import jax
import jax.numpy as jnp
from jax import lax
from jax.experimental import pallas as pl
from jax.experimental.pallas import tpu as pltpu
from jax.experimental.pallas import tpu_sc as plsc

N = 10000
DI = 128
DH = 128
DO = 40
E = 320000

NC = 2
NS = 16
NW = NC * NS
CH = 128
NCH = E // CH
CPW = NCH // NW
NTAIL = NCH - CPW * NW
HC = CPW // 2
RPW = N // NS

_mesh = plsc.VectorSubcoreMesh(core_axis_name="c", subcore_axis_name="s")
_sc_params = pltpu.CompilerParams(use_tc_tiling_on_sc=False)


def _zero_rows(buf, d):
    zero = jnp.zeros((16,), jnp.float32)

    def body(i, _):
        for t in range(-(-d // 16)):
            buf[i, pl.ds(min(16 * t, d - 16), 16)] = zero
        return ()

    lax.fori_loop(0, CH, body, (), unroll=False)


def _zero_spmem_slice(buf, table, sid, d):
    base = sid * RPW
    for k in range(4):
        pltpu.sync_copy(buf, table.at[pl.ds(base + k * CH, CH)])
    pltpu.sync_copy(buf.at[pl.ds(0, RPW - 4 * CH)],
                    table.at[pl.ds(base + 4 * CH, RPW - 4 * CH)])


def _make_deg():
    @functools.partial(
        pl.kernel,
        out_type=jax.ShapeDtypeStruct((NC, N, 16), jnp.float32),
        mesh=_mesh,
        scratch_types=[
            pltpu.VMEM((HC, CH), jnp.int32),
            pltpu.VMEM((1, CH), jnp.int32),
            pltpu.VMEM((CH, 16), jnp.float32),
            pltpu.VMEM((CH, 16), jnp.float32),
            pltpu.VMEM_SHARED((N, 16), jnp.float32),
        ],
        compiler_params=_sc_params,
    )
    def deg_kernel(e_hbm, out_hbm, dst_v, tdst_v, ones_v, zbuf, deg_sp):
        cid = lax.axis_index("c")
        sid = lax.axis_index("s")
        wid = cid * NS + sid
        one = jnp.ones((16,), jnp.float32)

        def fill_ones(i, _):
            ones_v[i] = one
            return ()

        lax.fori_loop(0, CH, fill_ones, (), unroll=False)
        _zero_rows(zbuf, 16)
        _zero_spmem_slice(zbuf, deg_sp, sid, 16)
        plsc.subcore_barrier()

        for h in range(2):
            pltpu.sync_copy(e_hbm.at[1, pl.ds(wid * CPW + h * HC, HC)], dst_v)

            def body(j, _):
                pltpu.sync_copy(ones_v, deg_sp.at[dst_v.at[j]], add=True)
                return ()

            lax.fori_loop(0, HC, body, (), unroll=False)

        @pl.when(wid < NTAIL)
        def _():
            pltpu.sync_copy(e_hbm.at[1, pl.ds(NW * CPW + wid, 1)], tdst_v)
            pltpu.sync_copy(ones_v, deg_sp.at[tdst_v.at[0]], add=True)

        plsc.subcore_barrier()
        pltpu.sync_copy(deg_sp.at[pl.ds(sid * RPW, RPW)],
                        out_hbm.at[cid, pl.ds(sid * RPW, RPW)])

    return deg_kernel


def _make_agg(d):
    @functools.partial(
        pl.kernel,
        out_type=jax.ShapeDtypeStruct((NC, N, d), jnp.float32),
        mesh=_mesh,
        scratch_types=[
            pltpu.VMEM((HC, CH), jnp.int32),
            pltpu.VMEM((HC, CH), jnp.int32),
            pltpu.VMEM((1, CH), jnp.int32),
            pltpu.VMEM((1, CH), jnp.int32),
            pltpu.VMEM((CH, d), jnp.float32),
            pltpu.VMEM((CH, d), jnp.float32),
            pltpu.VMEM_SHARED((N, d), jnp.float32),
            pltpu.SemaphoreType.DMA,
            pltpu.SemaphoreType.DMA,
        ],
        compiler_params=_sc_params,
    )
    def agg_kernel(table_hbm, e_hbm, out_hbm, src_v, dst_v, tsrc_v, tdst_v,
                   buf0, buf1, acc_sp, g0, g1):
        cid = lax.axis_index("c")
        sid = lax.axis_index("s")
        wid = cid * NS + sid

        def gwait(sem, buf):
            pltpu.make_async_copy(table_hbm.at[pl.ds(0, CH)], buf, sem).wait()

        _zero_rows(buf0, d)
        _zero_spmem_slice(buf0, acc_sp, sid, d)
        plsc.subcore_barrier()

        for h in range(2):
            base = wid * CPW + h * HC
            pltpu.sync_copy(e_hbm.at[0, pl.ds(base, HC)], src_v)
            pltpu.sync_copy(e_hbm.at[1, pl.ds(base, HC)], dst_v)
            pltpu.async_copy(table_hbm.at[src_v.at[0]], buf0, g0)

            def body(jj, _):
                j = jj * 2
                pltpu.async_copy(table_hbm.at[src_v.at[j + 1]], buf1, g1)
                gwait(g0, buf0)
                pltpu.sync_copy(buf0, acc_sp.at[dst_v.at[j]], add=True)

                @pl.when(j + 2 < HC)
                def _():
                    pltpu.async_copy(table_hbm.at[src_v.at[j + 2]], buf0, g0)

                gwait(g1, buf1)
                pltpu.sync_copy(buf1, acc_sp.at[dst_v.at[j + 1]], add=True)
                return ()

            lax.fori_loop(0, (HC - 1) // 2, body, (), unroll=False)
            gwait(g0, buf0)
            pltpu.sync_copy(buf0, acc_sp.at[dst_v.at[HC - 1]], add=True)

        @pl.when(wid < NTAIL)
        def _():
            pltpu.sync_copy(e_hbm.at[0, pl.ds(NW * CPW + wid, 1)], tsrc_v)
            pltpu.sync_copy(e_hbm.at[1, pl.ds(NW * CPW + wid, 1)], tdst_v)
            pltpu.async_copy(table_hbm.at[tsrc_v.at[0]], buf0, g0)
            gwait(g0, buf0)
            pltpu.sync_copy(buf0, acc_sp.at[tdst_v.at[0]], add=True)

        plsc.subcore_barrier()
        pltpu.sync_copy(acc_sp.at[pl.ds(sid * RPW, RPW)],
                        out_hbm.at[cid, pl.ds(sid * RPW, RPW)])

    return agg_kernel


_deg_call = _make_deg()
_agg128 = _make_agg(DH)
_agg40 = _make_agg(DO)

BR = 2000


def _mm_body(x_ref, w1_ref, h_ref):
    h_ref[...] = jnp.dot(x_ref[...], w1_ref[...],
                         preferred_element_type=jnp.float32)


def _dis(deg_ref):
    deg = deg_ref[0, :, 0:1] + deg_ref[1, :, 0:1] + 1.0
    return lax.rsqrt(deg)


def _scale_body(deg_ref, h_ref, hs_ref):
    hs_ref[...] = _dis(deg_ref) * h_ref[...]


def _tc2_body(deg_ref, acc_ref, hs1_ref, b1_ref, w2_ref, hs2_ref):
    dis = _dis(deg_ref)
    agg = (acc_ref[0] + acc_ref[1] + hs1_ref[...]) * dis
    m1 = jnp.maximum(agg + b1_ref[...], 0.0)
    h2 = jnp.dot(m1, w2_ref[...], preferred_element_type=jnp.float32)
    hs2_ref[...] = dis * h2


def _tc3_body(deg_ref, acc_ref, hs2_ref, b2_ref, out_ref):
    agg = (acc_ref[0] + acc_ref[1] + hs2_ref[...]) * _dis(deg_ref)
    out_ref[...] = agg + b2_ref[...]


def kernel(x, edge_index, W1, b1, W2, b2):
    e3 = edge_index.astype(jnp.int32).reshape(2, NCH, CH)

    grid = (N // BR,)
    row = lambda w: pl.BlockSpec((BR, w), lambda i: (i, 0))
    deg_spec = pl.BlockSpec((2, BR, 16), lambda i: (0, i, 0))
    acc_spec = lambda w: pl.BlockSpec((2, BR, w), lambda i: (0, i, 0))
    full = lambda a, b: pl.BlockSpec((a, b), lambda i: (0, 0))

    degs = _deg_call(e3)

    h1 = pl.pallas_call(
        _mm_body,
        grid=grid,
        in_specs=[row(DI), full(DI, DH)],
        out_specs=row(DH),
        out_shape=jax.ShapeDtypeStruct((N, DH), jnp.float32),
    )(x, W1)

    hs1 = pl.pallas_call(
        _scale_body,
        grid=grid,
        in_specs=[deg_spec, row(DH)],
        out_specs=row(DH),
        out_shape=jax.ShapeDtypeStruct((N, DH), jnp.float32),
    )(degs, h1)

    acc1 = _agg128(hs1, e3)

    hs2 = pl.pallas_call(
        _tc2_body,
        grid=grid,
        in_specs=[deg_spec, acc_spec(DH), row(DH), full(1, DH), full(DH, DO)],
        out_specs=row(DO),
        out_shape=jax.ShapeDtypeStruct((N, DO), jnp.float32),
    )(degs, acc1, hs1, b1.reshape(1, DH), W2)

    acc2 = _agg40(hs2, e3)

    out = pl.pallas_call(
        _tc3_body,
        grid=grid,
        in_specs=[deg_spec, acc_spec(DO), row(DO), full(1, DO)],
        out_specs=row(DO),
        out_shape=jax.ShapeDtypeStruct((N, DO), jnp.float32),
    )(degs, acc2, hs2, b2.reshape(1, DO))

    return out


# speedup vs baseline: 38.9752x; 1.0306x over previous
import functools

import jax
import jax.numpy as jnp
from jax import lax
from jax.experimental import pallas as pl
from jax.experimental.pallas import tpu as pltpu
from jax.experimental.pallas import tpu_sc as plsc

N = 10000
DI = 128
DH = 128
DO = 40
E = 320000

NC = 2
NS = 16
NW = NC * NS
CH = 128
NCH = E // CH
CPW = NCH // NW
NTAIL = NCH - CPW * NW
HC = CPW // 2
RPW = N // NS

_mesh = plsc.VectorSubcoreMesh(core_axis_name="c", subcore_axis_name="s")
_sc_params = pltpu.CompilerParams(use_tc_tiling_on_sc=False)


def _zero_rows(buf, d):
    zero = jnp.zeros((16,), jnp.float32)

    def body(i, _):
        for t in range(-(-d // 16)):
            buf[i, pl.ds(min(16 * t, d - 16), 16)] = zero
        return ()

    lax.fori_loop(0, CH, body, (), unroll=False)


def _zero_spmem_slice(buf, table, sid, d):
    base = sid * RPW
    for k in range(4):
        pltpu.sync_copy(buf, table.at[pl.ds(base + k * CH, CH)])
    pltpu.sync_copy(buf.at[pl.ds(0, RPW - 4 * CH)],
                    table.at[pl.ds(base + 4 * CH, RPW - 4 * CH)])


def _make_deg():
    @functools.partial(
        pl.kernel,
        out_type=jax.ShapeDtypeStruct((NC, N, 16), jnp.float32),
        mesh=_mesh,
        scratch_types=[
            pltpu.VMEM((HC, CH), jnp.int32),
            pltpu.VMEM((1, CH), jnp.int32),
            pltpu.VMEM((CH, 16), jnp.float32),
            pltpu.VMEM((CH, 16), jnp.float32),
            pltpu.VMEM_SHARED((N, 16), jnp.float32),
            pltpu.SemaphoreType.DMA,
        ],
        compiler_params=_sc_params,
    )
    def deg_kernel(e_hbm, out_hbm, dst_v, tdst_v, ones_v, zbuf, deg_sp, dsem):
        cid = lax.axis_index("c")
        sid = lax.axis_index("s")
        wid = cid * NS + sid
        one = jnp.ones((16,), jnp.float32)

        def fill_ones(i, _):
            ones_v[i] = one
            return ()

        lax.fori_loop(0, CH, fill_ones, (), unroll=False)
        _zero_rows(zbuf, 16)
        _zero_spmem_slice(zbuf, deg_sp, sid, 16)
        plsc.subcore_barrier()

        def drain_one():
            pltpu.make_async_copy(out_hbm.at[0, pl.ds(0, CH)], zbuf, dsem).wait()

        for h in range(2):
            pltpu.sync_copy(e_hbm.at[1, pl.ds(wid * CPW + h * HC, HC)], dst_v)

            def body(j, _):
                pltpu.async_copy(ones_v, deg_sp.at[dst_v.at[j]], dsem, add=True)

                @pl.when(j >= 6)
                def _():
                    drain_one()

                return ()

            lax.fori_loop(0, HC, body, (), unroll=False)
            for _ in range(6):
                drain_one()

        @pl.when(wid < NTAIL)
        def _():
            pltpu.sync_copy(e_hbm.at[1, pl.ds(NW * CPW + wid, 1)], tdst_v)
            pltpu.sync_copy(ones_v, deg_sp.at[tdst_v.at[0]], add=True)

        plsc.subcore_barrier()
        pltpu.sync_copy(deg_sp.at[pl.ds(sid * RPW, RPW)],
                        out_hbm.at[cid, pl.ds(sid * RPW, RPW)])

    return deg_kernel


def _make_agg(d, paired):
    nbuf = 4 if paired else 2
    scratch = [
        pltpu.VMEM((HC, CH), jnp.int32),
        pltpu.VMEM((HC, CH), jnp.int32),
        pltpu.VMEM((1, CH), jnp.int32),
        pltpu.VMEM((1, CH), jnp.int32),
    ] + [pltpu.VMEM((CH, d), jnp.float32)] * nbuf + [
        pltpu.VMEM_SHARED((N, d), jnp.float32),
        pltpu.SemaphoreType.DMA,
        pltpu.SemaphoreType.DMA,
        pltpu.SemaphoreType.DMA,
        pltpu.SemaphoreType.DMA,
    ]

    @functools.partial(
        pl.kernel,
        out_type=jax.ShapeDtypeStruct((NC, N, d), jnp.float32),
        mesh=_mesh,
        scratch_types=scratch,
        compiler_params=_sc_params,
    )
    def agg_kernel(table_hbm, e_hbm, out_hbm, src_v, dst_v, tsrc_v, tdst_v,
                   *bufs_and_sems):
        bufs = bufs_and_sems[:nbuf]
        acc_sp = bufs_and_sems[nbuf]
        g0, g1, s0, s1 = bufs_and_sems[nbuf + 1:]
        buf0, buf1 = bufs[0], bufs[1]
        cid = lax.axis_index("c")
        sid = lax.axis_index("s")
        wid = cid * NS + sid

        def wait(sem, n=1):
            for _ in range(n):
                pltpu.make_async_copy(table_hbm.at[pl.ds(0, CH)], buf0,
                                      sem).wait()

        def gather(c, buf, sem):
            pltpu.async_copy(table_hbm.at[src_v.at[c]], buf, sem)

        def scat_sync(c, buf):
            pltpu.sync_copy(buf, acc_sp.at[dst_v.at[c]], add=True)

        def scat_async(c, buf, sem):
            pltpu.async_copy(buf, acc_sp.at[dst_v.at[c]], sem, add=True)

        _zero_rows(buf0, d)
        _zero_spmem_slice(buf0, acc_sp, sid, d)
        plsc.subcore_barrier()

        for h in range(2):
            base = wid * CPW + h * HC
            pltpu.sync_copy(e_hbm.at[0, pl.ds(base, HC)], src_v)
            pltpu.sync_copy(e_hbm.at[1, pl.ds(base, HC)], dst_v)

            if not paired:
                gather(0, buf0, g0)

                def body(jj, _):
                    j = jj * 2
                    gather(j + 1, buf1, g1)
                    wait(g0)
                    scat_sync(j, buf0)

                    @pl.when(j + 2 < HC)
                    def _():
                        gather(j + 2, buf0, g0)

                    wait(g1)
                    scat_sync(j + 1, buf1)
                    return ()

                lax.fori_loop(0, (HC - 1) // 2, body, (), unroll=False)
                wait(g0)
                scat_sync(HC - 1, buf0)
            else:
                b0a, b0b, b1a, b1b = bufs
                gather(0, b0a, g0)
                gather(1, b0b, g0)
                gather(2, b1a, g1)
                gather(3, b1b, g1)

                def body(jj, _):
                    c = jj * 4
                    wait(g0, 2)
                    scat_async(c, b0a, s0)
                    scat_async(c + 1, b0b, s0)
                    wait(g1, 2)
                    scat_async(c + 2, b1a, s1)
                    scat_async(c + 3, b1b, s1)
                    wait(s0, 2)
                    gather(c + 4, b0a, g0)
                    gather(c + 5, b0b, g0)
                    wait(s1, 2)
                    gather(c + 6, b1a, g1)

                    @pl.when(c + 7 < HC)
                    def _():
                        gather(c + 7, b1b, g1)

                    return ()

                lax.fori_loop(0, (HC - 3) // 4, body, (), unroll=False)
                wait(g0, 2)
                scat_sync(HC - 3, b0a)
                scat_sync(HC - 2, b0b)
                wait(g1)
                scat_sync(HC - 1, b1a)

        @pl.when(wid < NTAIL)
        def _():
            pltpu.sync_copy(e_hbm.at[0, pl.ds(NW * CPW + wid, 1)], tsrc_v)
            pltpu.sync_copy(e_hbm.at[1, pl.ds(NW * CPW + wid, 1)], tdst_v)
            pltpu.async_copy(table_hbm.at[tsrc_v.at[0]], buf0, g0)
            wait(g0)
            pltpu.sync_copy(buf0, acc_sp.at[tdst_v.at[0]], add=True)

        plsc.subcore_barrier()
        pltpu.sync_copy(acc_sp.at[pl.ds(sid * RPW, RPW)],
                        out_hbm.at[cid, pl.ds(sid * RPW, RPW)])

    return agg_kernel


_deg_call = _make_deg()
_agg128 = _make_agg(DH, paired=False)
_agg40 = _make_agg(DO, paired=True)

BR = 2000


def _mm_body(x_ref, w1_ref, h_ref):
    h_ref[...] = jnp.dot(x_ref[...], w1_ref[...],
                         preferred_element_type=jnp.float32)


def _dis(deg_ref):
    deg = deg_ref[0, :, 0:1] + deg_ref[1, :, 0:1] + 1.0
    return lax.rsqrt(deg)


def _scale_body(deg_ref, h_ref, hs_ref):
    hs_ref[...] = _dis(deg_ref) * h_ref[...]


def _tc2_body(deg_ref, acc_ref, hs1_ref, b1_ref, w2_ref, hs2_ref):
    dis = _dis(deg_ref)
    agg = (acc_ref[0] + acc_ref[1] + hs1_ref[...]) * dis
    m1 = jnp.maximum(agg + b1_ref[...], 0.0)
    h2 = jnp.dot(m1, w2_ref[...], preferred_element_type=jnp.float32)
    hs2_ref[...] = dis * h2


def _tc3_body(deg_ref, acc_ref, hs2_ref, b2_ref, out_ref):
    agg = (acc_ref[0] + acc_ref[1] + hs2_ref[...]) * _dis(deg_ref)
    out_ref[...] = agg + b2_ref[...]


def kernel(x, edge_index, W1, b1, W2, b2):
    e3 = edge_index.astype(jnp.int32).reshape(2, NCH, CH)

    grid = (N // BR,)
    row = lambda w: pl.BlockSpec((BR, w), lambda i: (i, 0))
    deg_spec = pl.BlockSpec((2, BR, 16), lambda i: (0, i, 0))
    acc_spec = lambda w: pl.BlockSpec((2, BR, w), lambda i: (0, i, 0))
    full = lambda a, b: pl.BlockSpec((a, b), lambda i: (0, 0))

    degs = _deg_call(e3)

    h1 = pl.pallas_call(
        _mm_body,
        grid=grid,
        in_specs=[row(DI), full(DI, DH)],
        out_specs=row(DH),
        out_shape=jax.ShapeDtypeStruct((N, DH), jnp.float32),
    )(x, W1)

    hs1 = pl.pallas_call(
        _scale_body,
        grid=grid,
        in_specs=[deg_spec, row(DH)],
        out_specs=row(DH),
        out_shape=jax.ShapeDtypeStruct((N, DH), jnp.float32),
    )(degs, h1)

    acc1 = _agg128(hs1, e3)

    hs2 = pl.pallas_call(
        _tc2_body,
        grid=grid,
        in_specs=[deg_spec, acc_spec(DH), row(DH), full(1, DH), full(DH, DO)],
        out_specs=row(DO),
        out_shape=jax.ShapeDtypeStruct((N, DO), jnp.float32),
    )(degs, acc1, hs1, b1.reshape(1, DH), W2)

    acc2 = _agg40(hs2, e3)

    out = pl.pallas_call(
        _tc3_body,
        grid=grid,
        in_specs=[deg_spec, acc_spec(DO), row(DO), full(1, DO)],
        out_specs=row(DO),
        out_shape=jax.ShapeDtypeStruct((N, DO), jnp.float32),
    )(degs, acc2, hs2, b2.reshape(1, DO))

    return out


# speedup vs baseline: 40.6393x; 1.0427x over previous
import functools

import jax
import jax.numpy as jnp
from jax import lax
from jax.experimental import pallas as pl
from jax.experimental.pallas import tpu as pltpu
from jax.experimental.pallas import tpu_sc as plsc

N = 10000
DI = 128
DH = 128
DO = 40
E = 320000

NC = 2
NS = 16
NW = NC * NS
CH = 128
NCH = E // CH
CPW = NCH // NW
NTAIL = NCH - CPW * NW
HC = CPW // 2
RPW = N // NS

_mesh = plsc.VectorSubcoreMesh(core_axis_name="c", subcore_axis_name="s")
_sc_params = pltpu.CompilerParams(use_tc_tiling_on_sc=False)


def _zero_rows(buf, d):
    zero = jnp.zeros((16,), jnp.float32)

    def body(i, _):
        for t in range(-(-d // 16)):
            buf[i, pl.ds(min(16 * t, d - 16), 16)] = zero
        return ()

    lax.fori_loop(0, CH, body, (), unroll=False)


def _zero_spmem_slice(buf, table, sid, d):
    base = sid * RPW
    for k in range(4):
        pltpu.sync_copy(buf, table.at[pl.ds(base + k * CH, CH)])
    pltpu.sync_copy(buf.at[pl.ds(0, RPW - 4 * CH)],
                    table.at[pl.ds(base + 4 * CH, RPW - 4 * CH)])


def _make_deg():
    @functools.partial(
        pl.kernel,
        out_type=jax.ShapeDtypeStruct((NC, N, 16), jnp.float32),
        mesh=_mesh,
        scratch_types=[
            pltpu.VMEM((HC, CH), jnp.int32),
            pltpu.VMEM((1, CH), jnp.int32),
            pltpu.VMEM((CH, 16), jnp.float32),
            pltpu.VMEM((CH, 16), jnp.float32),
            pltpu.VMEM_SHARED((N, 16), jnp.float32),
            pltpu.SemaphoreType.DMA,
        ],
        compiler_params=_sc_params,
    )
    def deg_kernel(e_hbm, out_hbm, dst_v, tdst_v, ones_v, zbuf, deg_sp, dsem):
        cid = lax.axis_index("c")
        sid = lax.axis_index("s")
        wid = cid * NS + sid
        one = jnp.ones((16,), jnp.float32)

        def fill_ones(i, _):
            ones_v[i] = one
            return ()

        lax.fori_loop(0, CH, fill_ones, (), unroll=False)
        _zero_rows(zbuf, 16)
        _zero_spmem_slice(zbuf, deg_sp, sid, 16)
        plsc.subcore_barrier()

        def drain_one():
            pltpu.make_async_copy(out_hbm.at[0, pl.ds(0, CH)], zbuf, dsem).wait()

        for h in range(2):
            pltpu.sync_copy(e_hbm.at[1, pl.ds(wid * CPW + h * HC, HC)], dst_v)

            def body(j, _):
                pltpu.async_copy(ones_v, deg_sp.at[dst_v.at[j]], dsem, add=True)

                @pl.when(j >= 12)
                def _():
                    drain_one()

                return ()

            lax.fori_loop(0, HC, body, (), unroll=False)
            for _ in range(12):
                drain_one()

        @pl.when(wid < NTAIL)
        def _():
            pltpu.sync_copy(e_hbm.at[1, pl.ds(NW * CPW + wid, 1)], tdst_v)
            pltpu.sync_copy(ones_v, deg_sp.at[tdst_v.at[0]], add=True)

        plsc.subcore_barrier()
        pltpu.sync_copy(deg_sp.at[pl.ds(sid * RPW, RPW)],
                        out_hbm.at[cid, pl.ds(sid * RPW, RPW)])

    return deg_kernel


def _make_agg(d, paired, out_minor=None):
    om = d if out_minor is None else out_minor
    nbuf = 4 if paired else 2
    scratch = [
        pltpu.VMEM((HC, CH), jnp.int32),
        pltpu.VMEM((HC, CH), jnp.int32),
        pltpu.VMEM((1, CH), jnp.int32),
        pltpu.VMEM((1, CH), jnp.int32),
    ] + [pltpu.VMEM((CH, d), jnp.float32)] * nbuf + [
        pltpu.VMEM_SHARED((N, d), jnp.float32),
        pltpu.SemaphoreType.DMA,
        pltpu.SemaphoreType.DMA,
        pltpu.SemaphoreType.DMA,
        pltpu.SemaphoreType.DMA,
    ]

    @functools.partial(
        pl.kernel,
        out_type=jax.ShapeDtypeStruct((NC, N, om), jnp.float32),
        mesh=_mesh,
        scratch_types=scratch,
        compiler_params=_sc_params,
    )
    def agg_kernel(table_hbm, e_hbm, out_hbm, src_v, dst_v, tsrc_v, tdst_v,
                   *bufs_and_sems):
        bufs = bufs_and_sems[:nbuf]
        acc_sp = bufs_and_sems[nbuf]
        g0, g1, s0, s1 = bufs_and_sems[nbuf + 1:]
        buf0, buf1 = bufs[0], bufs[1]
        cid = lax.axis_index("c")
        sid = lax.axis_index("s")
        wid = cid * NS + sid

        def wait(sem, n=1):
            for _ in range(n):
                pltpu.make_async_copy(table_hbm.at[pl.ds(0, CH)], buf0,
                                      sem).wait()

        def gather(c, buf, sem):
            pltpu.async_copy(table_hbm.at[src_v.at[c]], buf, sem)

        def scat_sync(c, buf):
            pltpu.sync_copy(buf, acc_sp.at[dst_v.at[c]], add=True)

        def scat_async(c, buf, sem):
            pltpu.async_copy(buf, acc_sp.at[dst_v.at[c]], sem, add=True)

        _zero_rows(buf0, d)
        _zero_spmem_slice(buf0, acc_sp, sid, d)
        plsc.subcore_barrier()

        for h in range(2):
            base = wid * CPW + h * HC
            pltpu.sync_copy(e_hbm.at[0, pl.ds(base, HC)], src_v)
            pltpu.sync_copy(e_hbm.at[1, pl.ds(base, HC)], dst_v)

            if not paired:
                gather(0, buf0, g0)

                def body(jj, _):
                    j = jj * 2
                    gather(j + 1, buf1, g1)
                    wait(g0)
                    scat_sync(j, buf0)

                    @pl.when(j + 2 < HC)
                    def _():
                        gather(j + 2, buf0, g0)

                    wait(g1)
                    scat_sync(j + 1, buf1)
                    return ()

                lax.fori_loop(0, (HC - 1) // 2, body, (), unroll=False)
                wait(g0)
                scat_sync(HC - 1, buf0)
            else:
                b0a, b0b, b1a, b1b = bufs
                gather(0, b0a, g0)
                gather(1, b0b, g0)
                gather(2, b1a, g1)
                gather(3, b1b, g1)

                def body(jj, _):
                    c = jj * 4
                    wait(g0, 2)
                    scat_async(c, b0a, s0)
                    scat_async(c + 1, b0b, s0)
                    wait(g1, 2)
                    scat_async(c + 2, b1a, s1)
                    scat_async(c + 3, b1b, s1)
                    wait(s0, 2)
                    gather(c + 4, b0a, g0)
                    gather(c + 5, b0b, g0)
                    wait(s1, 2)
                    gather(c + 6, b1a, g1)

                    @pl.when(c + 7 < HC)
                    def _():
                        gather(c + 7, b1b, g1)

                    return ()

                lax.fori_loop(0, (HC - 3) // 4, body, (), unroll=False)
                wait(g0, 2)
                scat_sync(HC - 3, b0a)
                scat_sync(HC - 2, b0b)
                wait(g1)
                scat_sync(HC - 1, b1a)

        @pl.when(wid < NTAIL)
        def _():
            pltpu.sync_copy(e_hbm.at[0, pl.ds(NW * CPW + wid, 1)], tsrc_v)
            pltpu.sync_copy(e_hbm.at[1, pl.ds(NW * CPW + wid, 1)], tdst_v)
            pltpu.async_copy(table_hbm.at[tsrc_v.at[0]], buf0, g0)
            wait(g0)
            pltpu.sync_copy(buf0, acc_sp.at[tdst_v.at[0]], add=True)

        plsc.subcore_barrier()
        if om == d:
            pltpu.sync_copy(acc_sp.at[pl.ds(sid * RPW, RPW)],
                            out_hbm.at[cid, pl.ds(sid * RPW, RPW)])
        else:
            pltpu.sync_copy(
                acc_sp.at[pl.ds(sid * RPW, RPW)],
                out_hbm.at[cid, pl.ds(sid * RPW, RPW), pl.ds(0, d)])

    return agg_kernel


_deg_call = _make_deg()
_agg128 = _make_agg(DH, paired=False)
_agg40 = _make_agg(DO, paired=True, out_minor=128)

BR = 2000


def _mm_body(x_ref, w1_ref, h_ref):
    h_ref[...] = jnp.dot(x_ref[...], w1_ref[...],
                         preferred_element_type=jnp.float32)


def _dis(deg_ref):
    deg = deg_ref[0, :, 0:1] + deg_ref[1, :, 0:1] + 1.0
    return lax.rsqrt(deg)


def _scale_body(deg_ref, h_ref, hs_ref):
    hs_ref[...] = _dis(deg_ref) * h_ref[...]


def _tc2_body(deg_ref, acc_ref, hs1_ref, b1_ref, w2_ref, hs2_ref):
    dis = _dis(deg_ref)
    agg = (acc_ref[0] + acc_ref[1] + hs1_ref[...]) * dis
    m1 = jnp.maximum(agg + b1_ref[...], 0.0)
    h2 = jnp.dot(m1, w2_ref[...], preferred_element_type=jnp.float32)
    hs2_ref[...] = dis * h2


def _tc3_body(deg_ref, acc_ref, hs2_ref, b2_ref, out_ref):
    acc = acc_ref[0, :, 0:DO] + acc_ref[1, :, 0:DO]
    agg = (acc + hs2_ref[...]) * _dis(deg_ref)
    out_ref[...] = agg + b2_ref[...]


def kernel(x, edge_index, W1, b1, W2, b2):
    e3 = edge_index.astype(jnp.int32).reshape(2, NCH, CH)

    grid = (N // BR,)
    row = lambda w: pl.BlockSpec((BR, w), lambda i: (i, 0))
    deg_spec = pl.BlockSpec((2, BR, 16), lambda i: (0, i, 0))
    acc_spec = lambda w: pl.BlockSpec((2, BR, w), lambda i: (0, i, 0))
    full = lambda a, b: pl.BlockSpec((a, b), lambda i: (0, 0))

    degs = _deg_call(e3)

    h1 = pl.pallas_call(
        _mm_body,
        grid=grid,
        in_specs=[row(DI), full(DI, DH)],
        out_specs=row(DH),
        out_shape=jax.ShapeDtypeStruct((N, DH), jnp.float32),
    )(x, W1)

    hs1 = pl.pallas_call(
        _scale_body,
        grid=grid,
        in_specs=[deg_spec, row(DH)],
        out_specs=row(DH),
        out_shape=jax.ShapeDtypeStruct((N, DH), jnp.float32),
    )(degs, h1)

    acc1 = _agg128(hs1, e3)

    hs2 = pl.pallas_call(
        _tc2_body,
        grid=grid,
        in_specs=[deg_spec, acc_spec(DH), row(DH), full(1, DH), full(DH, DO)],
        out_specs=row(DO),
        out_shape=jax.ShapeDtypeStruct((N, DO), jnp.float32),
    )(degs, acc1, hs1, b1.reshape(1, DH), W2)

    acc2 = _agg40(hs2, e3)

    out = pl.pallas_call(
        _tc3_body,
        grid=grid,
        in_specs=[deg_spec, acc_spec(DH), row(DO), full(1, DO)],
        out_specs=row(DO),
        out_shape=jax.ShapeDtypeStruct((N, DO), jnp.float32),
    )(degs, acc2, hs2, b2.reshape(1, DO))

    return out
